# baseline (speedup 1.0000x reference)
import jax
import jax.numpy as jnp
from jax import lax
from jax.experimental import pallas as pl
from jax.experimental.pallas import tpu as pltpu
from jax.experimental.pallas import tpu_sc as plsc

N_EDGES = 800000
D = 64
L = 16
NC = 2
NS = 16
NW = NC * NS
C = 640
NCHUNK = N_EDGES // C
TPW = -(-NCHUNK // NW)


def _sc_body(et_hbm, bond_hbm, stereo_hbm, conj_hbm, ring_hbm, out_hbm,
             tab15_v, bs_v, cr_v, ctab_v, idx_v, out_v):
    wid = lax.axis_index("s") * NC + lax.axis_index("c")

    pltpu.sync_copy(bond_hbm, tab15_v.at[pl.ds(0 * D, 5 * D)])
    pltpu.sync_copy(stereo_hbm, tab15_v.at[pl.ds(5 * D, 6 * D)])
    pltpu.sync_copy(conj_hbm, tab15_v.at[pl.ds(11 * D, 2 * D)])
    pltpu.sync_copy(ring_hbm, tab15_v.at[pl.ds(13 * D, 2 * D)])

    def bs_row(r2, _):
        i = r2 // 6
        j = r2 % 6
        for cg in range(D // L):
            o = cg * L
            bs_v[pl.ds(r2 * D + o, L)] = (tab15_v[pl.ds(i * D + o, L)]
                                          + tab15_v[pl.ds((5 + j) * D + o, L)])
        return 0

    lax.fori_loop(0, 30, bs_row, 0)

    for q in range(4):
        for cg in range(D // L):
            o = cg * L
            cr_v[pl.ds(q * D + o, L)] = (tab15_v[pl.ds((11 + q // 2) * D + o, L)]
                                         + tab15_v[pl.ds((13 + q % 2) * D + o, L)])

    def ctab_row(r, _):
        r2 = r // 4
        q = r % 4
        for cg in range(D // L):
            o = cg * L
            ctab_v[pl.ds(r * D + o, L)] = (bs_v[pl.ds(r2 * D + o, L)]
                                           + cr_v[pl.ds(q * D + o, L)])
        return 0

    lax.fori_loop(0, 120, ctab_row, 0)

    lanes = lax.iota(jnp.int32, L)

    def do_chunk(cid):
        base = cid * C
        pltpu.sync_copy(et_hbm.at[pl.ds(base * 4, C * 4)], idx_v)

        def group(g, _):
            e_off = (g * L + lanes) * 4
            i0 = plsc.load_gather(idx_v, [e_off])
            i1 = plsc.load_gather(idx_v, [e_off + 1])
            i2 = plsc.load_gather(idx_v, [e_off + 2])
            i3 = plsc.load_gather(idx_v, [e_off + 3])
            comb64 = (i0 * 24 + i1 * 4 + i2 * 2 + i3) * D
            row64 = (g * L + lanes) * D
            for col in range(D):
                vals = plsc.load_gather(ctab_v, [comb64 + col])
                plsc.store_scatter(out_v, [row64 + col], vals)
            return 0

        lax.fori_loop(0, C // L, group, 0)
        pltpu.sync_copy(out_v, out_hbm.at[pl.ds(base * D, C * D)])

    def chunk_loop(t, _):
        cid = wid + t * NW
        @pl.when(cid < NCHUNK)
        def _():
            do_chunk(cid)
        return 0

    lax.fori_loop(0, TPW, chunk_loop, 0)


@jax.jit
def _edge_encode(et_flat, bond_flat, stereo_flat, conj_flat, ring_flat):
    mesh = plsc.VectorSubcoreMesh(core_axis_name="c", subcore_axis_name="s")
    k = pl.kernel(
        _sc_body,
        out_type=jax.ShapeDtypeStruct((N_EDGES * D,), jnp.float32),
        mesh=mesh,
        compiler_params=pltpu.CompilerParams(needs_layout_passes=False,
                                             disable_bounds_checks=True),
        scratch_types=[
            pltpu.VMEM((16 * D,), jnp.float32),
            pltpu.VMEM((30 * D,), jnp.float32),
            pltpu.VMEM((4 * D,), jnp.float32),
            pltpu.VMEM((120 * D,), jnp.float32),
            pltpu.VMEM((C * 4,), jnp.int32),
            pltpu.VMEM((C * D,), jnp.float32),
        ],
    )
    return k(et_flat, bond_flat, stereo_flat, conj_flat, ring_flat)


def kernel(edge_tensor, bond_W, stereo_W, conj_W, ring_W):
    out = _edge_encode(edge_tensor.astype(jnp.int32).reshape(-1),
                       bond_W.reshape(-1), stereo_W.reshape(-1),
                       conj_W.reshape(-1), ring_W.reshape(-1))
    return out.reshape(N_EDGES, D)


# speedup vs baseline: 1.0407x; 1.0407x over previous
import jax
import jax.numpy as jnp
from jax import lax
from jax.experimental import pallas as pl
from jax.experimental.pallas import tpu as pltpu
from jax.experimental.pallas import tpu_sc as plsc

N_EDGES = 800000
D = 64
L = 16
NC = 2
NS = 16
NW = NC * NS
C = 640
G = 128
NCHUNK = N_EDGES // C
TPW = -(-NCHUNK // NW)


def _sc_body(et_hbm, bond_hbm, stereo_hbm, conj_hbm, ring_hbm, out_hbm,
             tab15_v, bs_v, cr_v, ctab_v, ctab_sh, idx_v, comb_v, rows_v, sem):
    sid = lax.axis_index("s")
    wid = sid * NC + lax.axis_index("c")

    def _build():
        pltpu.sync_copy(bond_hbm, tab15_v.at[pl.ds(0 * D, 5 * D)])
        pltpu.sync_copy(stereo_hbm, tab15_v.at[pl.ds(5 * D, 6 * D)])
        pltpu.sync_copy(conj_hbm, tab15_v.at[pl.ds(11 * D, 2 * D)])
        pltpu.sync_copy(ring_hbm, tab15_v.at[pl.ds(13 * D, 2 * D)])

        def bs_row(r2, _):
            i = r2 // 6
            j = r2 % 6
            for cg in range(D // L):
                o = cg * L
                bs_v[pl.ds(r2 * D + o, L)] = (tab15_v[pl.ds(i * D + o, L)]
                                              + tab15_v[pl.ds((5 + j) * D + o, L)])
            return 0

        lax.fori_loop(0, 30, bs_row, 0)

        for q in range(4):
            for cg in range(D // L):
                o = cg * L
                cr_v[pl.ds(q * D + o, L)] = (tab15_v[pl.ds((11 + q // 2) * D + o, L)]
                                             + tab15_v[pl.ds((13 + q % 2) * D + o, L)])

        def ctab_row(r, _):
            r2 = r // 4
            q = r % 4
            for cg in range(D // L):
                o = cg * L
                ctab_v[r, pl.ds(o, L)] = (bs_v[pl.ds(r2 * D + o, L)]
                                          + cr_v[pl.ds(q * D + o, L)])
            return 0

        lax.fori_loop(0, 120, ctab_row, 0)
        pltpu.sync_copy(ctab_v, ctab_sh)

    _build()
    plsc.subcore_barrier()

    lanes = lax.iota(jnp.int32, L)

    def do_chunk(cid):
        base = cid * C
        pltpu.sync_copy(et_hbm.at[pl.ds(base * 4, C * 4)], idx_v)

        def group_t(g, _):
            rows = g * L + lanes
            e_off = rows * 4
            i0 = plsc.load_gather(idx_v, [e_off])
            i1 = plsc.load_gather(idx_v, [e_off + 1])
            i2 = plsc.load_gather(idx_v, [e_off + 2])
            i3 = plsc.load_gather(idx_v, [e_off + 3])
            comb = i0 * 24 + i1 * 4 + i2 * 2 + i3
            for col in range(D):
                cvec = jnp.full((L,), col, jnp.int32)
                vals = plsc.load_gather(ctab_v, [comb, cvec])
                plsc.store_scatter(rows_v, [rows, cvec], vals)
            return 0

        lax.fori_loop(0, C // L, group_t, 0)

        pltpu.sync_copy(rows_v, out_hbm.at[pl.ds(base, C)])

    def chunk_loop(t, _):
        cid = wid + t * NW
        @pl.when(cid < NCHUNK)
        def _():
            do_chunk(cid)
        return 0

    lax.fori_loop(0, TPW, chunk_loop, 0)


@jax.jit
def _edge_encode(et_flat, bond_flat, stereo_flat, conj_flat, ring_flat):
    mesh = plsc.VectorSubcoreMesh(core_axis_name="c", subcore_axis_name="s")
    k = pl.kernel(
        _sc_body,
        out_type=jax.ShapeDtypeStruct((N_EDGES, D), jnp.float32),
        mesh=mesh,
        compiler_params=pltpu.CompilerParams(needs_layout_passes=False,
                                             disable_bounds_checks=True),
        scratch_types=[
            pltpu.VMEM((16 * D,), jnp.float32),
            pltpu.VMEM((30 * D,), jnp.float32),
            pltpu.VMEM((4 * D,), jnp.float32),
            pltpu.VMEM((120, D), jnp.float32),
            pltpu.VMEM_SHARED((120, D), jnp.float32),
            pltpu.VMEM((C * 4,), jnp.int32),
            pltpu.VMEM((C // G, G), jnp.int32),
            pltpu.VMEM((C, D), jnp.float32),
            pltpu.SemaphoreType.DMA,
        ],
    )
    return k(et_flat, bond_flat, stereo_flat, conj_flat, ring_flat)


def kernel(edge_tensor, bond_W, stereo_W, conj_W, ring_W):
    return _edge_encode(edge_tensor.astype(jnp.int32).reshape(-1),
                        bond_W.reshape(-1), stereo_W.reshape(-1),
                        conj_W.reshape(-1), ring_W.reshape(-1))


# speedup vs baseline: 2.0971x; 2.0150x over previous
import jax
import jax.numpy as jnp
from jax import lax
from jax.experimental import pallas as pl
from jax.experimental.pallas import tpu as pltpu
from jax.experimental.pallas import tpu_sc as plsc

N_EDGES = 800000
D = 64
L = 16
NC = 2
NS = 16
NW = NC * NS
C = 640
G = 128
NCHUNK = N_EDGES // C
TPW = -(-NCHUNK // NW)


def _sc_body(et_hbm, bond_hbm, stereo_hbm, conj_hbm, ring_hbm, out_hbm,
             tab15_v, bs_v, cr_v, ctab_v, ctab_sh, idx_v, comb0, comb1,
             comb2, comb3, comb4, rows_v, sem):
    comb_k = [comb0, comb1, comb2, comb3, comb4]
    sid = lax.axis_index("s")
    wid = sid * NC + lax.axis_index("c")

    def _build():
        pltpu.sync_copy(bond_hbm, tab15_v.at[pl.ds(0 * D, 5 * D)])
        pltpu.sync_copy(stereo_hbm, tab15_v.at[pl.ds(5 * D, 6 * D)])
        pltpu.sync_copy(conj_hbm, tab15_v.at[pl.ds(11 * D, 2 * D)])
        pltpu.sync_copy(ring_hbm, tab15_v.at[pl.ds(13 * D, 2 * D)])

        def bs_row(r2, _):
            i = r2 // 6
            j = r2 % 6
            for cg in range(D // L):
                o = cg * L
                bs_v[pl.ds(r2 * D + o, L)] = (tab15_v[pl.ds(i * D + o, L)]
                                              + tab15_v[pl.ds((5 + j) * D + o, L)])
            return 0

        lax.fori_loop(0, 30, bs_row, 0)

        for q in range(4):
            for cg in range(D // L):
                o = cg * L
                cr_v[pl.ds(q * D + o, L)] = (tab15_v[pl.ds((11 + q // 2) * D + o, L)]
                                             + tab15_v[pl.ds((13 + q % 2) * D + o, L)])

        def ctab_row(r, _):
            r2 = r // 4
            q = r % 4
            for cg in range(D // L):
                o = cg * L
                ctab_v[r, pl.ds(o, L)] = (bs_v[pl.ds(r2 * D + o, L)]
                                          + cr_v[pl.ds(q * D + o, L)])
            return 0

        lax.fori_loop(0, 120, ctab_row, 0)
        pltpu.sync_copy(ctab_v, ctab_sh)

    _build()
    plsc.subcore_barrier()

    lanes = lax.iota(jnp.int32, L)

    def do_chunk(cid):
        base = cid * C
        pltpu.sync_copy(et_hbm.at[pl.ds(base * 4, C * 4)], idx_v)

        for k in range(C // G):
            def group_t(g2, _, k=k):
                g = k * (G // L) + g2
                e_off = (g * L + lanes) * 4
                i0 = plsc.load_gather(idx_v, [e_off])
                i1 = plsc.load_gather(idx_v, [e_off + 1])
                i2 = plsc.load_gather(idx_v, [e_off + 2])
                i3 = plsc.load_gather(idx_v, [e_off + 3])
                comb = (i0 * 24 + i1 * 4 + i2 * 2 + i3) * 4
                comb_k[k][pl.ds(g2 * L, L)] = comb
                return 0

            lax.fori_loop(0, G // L, group_t, 0)

        for k in range(C // G):
            pltpu.async_copy(ctab_sh.at[comb_k[k]],
                             rows_v.at[pl.ds(k * G, G)], sem)
        for k in range(C // G):
            pltpu.make_async_copy(ctab_sh.at[comb_k[k]],
                                  rows_v.at[pl.ds(k * G, G)], sem).wait()

        pltpu.sync_copy(rows_v, out_hbm.at[pl.ds(base, C)])

    def chunk_loop(t, _):
        cid = wid + t * NW
        @pl.when(cid < NCHUNK)
        def _():
            do_chunk(cid)
        return 0

    lax.fori_loop(0, TPW, chunk_loop, 0)


@jax.jit
def _edge_encode(et_flat, bond_flat, stereo_flat, conj_flat, ring_flat):
    mesh = plsc.VectorSubcoreMesh(core_axis_name="c", subcore_axis_name="s")
    k = pl.kernel(
        _sc_body,
        out_type=jax.ShapeDtypeStruct((N_EDGES, D), jnp.float32),
        mesh=mesh,
        compiler_params=pltpu.CompilerParams(needs_layout_passes=False,
                                             disable_bounds_checks=True),
        scratch_types=[
            pltpu.VMEM((16 * D,), jnp.float32),
            pltpu.VMEM((30 * D,), jnp.float32),
            pltpu.VMEM((4 * D,), jnp.float32),
            pltpu.VMEM((120, D), jnp.float32),
            pltpu.VMEM_SHARED((120, D), jnp.float32),
            pltpu.VMEM((C * 4,), jnp.int32),
            pltpu.VMEM((G,), jnp.int32),
            pltpu.VMEM((G,), jnp.int32),
            pltpu.VMEM((G,), jnp.int32),
            pltpu.VMEM((G,), jnp.int32),
            pltpu.VMEM((G,), jnp.int32),
            pltpu.VMEM((C, D), jnp.float32),
            pltpu.SemaphoreType.DMA,
        ],
    )
    return k(et_flat, bond_flat, stereo_flat, conj_flat, ring_flat)


def kernel(edge_tensor, bond_W, stereo_W, conj_W, ring_W):
    return _edge_encode(edge_tensor.astype(jnp.int32).reshape(-1),
                        bond_W.reshape(-1), stereo_W.reshape(-1),
                        conj_W.reshape(-1), ring_W.reshape(-1))


# speedup vs baseline: 2.9429x; 1.4033x over previous
import jax
import jax.numpy as jnp
from jax import lax
from jax.experimental import pallas as pl
from jax.experimental.pallas import tpu as pltpu
from jax.experimental.pallas import tpu_sc as plsc

N_EDGES = 800000
D = 64
L = 16
NC = 2
NS = 16
NW = NC * NS
EB = 128
C = 640
NB = C // EB
NCHUNK = N_EDGES // C
TPW = -(-NCHUNK // NW)
ROW_GR = 4


def _sc_body(et_hbm, bond_hbm, stereo_hbm, conj_hbm, ring_hbm, out_hbm,
             tab15_v, bs_v, cr_v, ctab_v, ctab_sh, idx_v,
             c0, c1, c2, c3, c4, rows_v, sem):
    comb_k = [c0, c1, c2, c3, c4]
    wid = lax.axis_index("s") * NC + lax.axis_index("c")

    pltpu.sync_copy(bond_hbm, tab15_v.at[pl.ds(0 * D, 5 * D)])
    pltpu.sync_copy(stereo_hbm, tab15_v.at[pl.ds(5 * D, 6 * D)])
    pltpu.sync_copy(conj_hbm, tab15_v.at[pl.ds(11 * D, 2 * D)])
    pltpu.sync_copy(ring_hbm, tab15_v.at[pl.ds(13 * D, 2 * D)])

    def bs_row(r2, _):
        i = r2 // 6
        j = r2 % 6
        for cg in range(D // L):
            o = cg * L
            bs_v[pl.ds(r2 * D + o, L)] = (tab15_v[pl.ds(i * D + o, L)]
                                          + tab15_v[pl.ds((5 + j) * D + o, L)])
        return 0

    lax.fori_loop(0, 30, bs_row, 0)

    for q in range(4):
        for cg in range(D // L):
            o = cg * L
            cr_v[pl.ds(q * D + o, L)] = (tab15_v[pl.ds((11 + q // 2) * D + o, L)]
                                         + tab15_v[pl.ds((13 + q % 2) * D + o, L)])

    def ctab_row(r, _):
        r2 = r // 4
        q = r % 4
        for cg in range(D // L):
            o = cg * L
            ctab_v[r, pl.ds(o, L)] = (bs_v[pl.ds(r2 * D + o, L)]
                                      + cr_v[pl.ds(q * D + o, L)])
        return 0

    lax.fori_loop(0, 120, ctab_row, 0)
    pltpu.sync_copy(ctab_v, ctab_sh)
    plsc.subcore_barrier()

    def do_chunk(cid):
        for c in range(4):
            pltpu.sync_copy(et_hbm.at[pl.ds(c * N_EDGES + cid * C, C)],
                            idx_v.at[pl.ds(c * C, C)])

        for k in range(NB):
            def fill(l, _, k=k):
                o = k * EB + l * L
                i0 = idx_v[pl.ds(o, L)]
                i1 = idx_v[pl.ds(C + o, L)]
                i2 = idx_v[pl.ds(2 * C + o, L)]
                i3 = idx_v[pl.ds(3 * C + o, L)]
                comb_k[k][pl.ds(l * L, L)] = (i0 * 24 + i1 * 4 + i2 * 2
                                              + i3) * ROW_GR
                return 0

            lax.fori_loop(0, EB // L, fill, 0)

        for k in range(NB):
            pltpu.async_copy(ctab_sh.at[comb_k[k]],
                             rows_v.at[pl.ds(k * EB, EB)], sem)
        for k in range(NB):
            pltpu.make_async_copy(ctab_sh.at[comb_k[k]],
                                  rows_v.at[pl.ds(k * EB, EB)], sem).wait()

        pltpu.sync_copy(rows_v, out_hbm.at[pl.ds(cid * C, C)])

    def chunk_loop(t, _):
        cid = wid + t * NW
        @pl.when(cid < NCHUNK)
        def _():
            do_chunk(cid)
        return 0

    lax.fori_loop(0, TPW, chunk_loop, 0)


TR_BLK = 640


def _tr_body(x_ref, o_ref):
    o_ref[...] = x_ref[...].T


@jax.jit
def _edge_encode(et_flat, bond_flat, stereo_flat, conj_flat, ring_flat):
    mesh = plsc.VectorSubcoreMesh(core_axis_name="c", subcore_axis_name="s")
    k = pl.kernel(
        _sc_body,
        out_type=jax.ShapeDtypeStruct((N_EDGES, D), jnp.float32),
        mesh=mesh,
        compiler_params=pltpu.CompilerParams(needs_layout_passes=False,
                                             disable_bounds_checks=True),
        scratch_types=[
            pltpu.VMEM((16 * D,), jnp.float32),
            pltpu.VMEM((30 * D,), jnp.float32),
            pltpu.VMEM((4 * D,), jnp.float32),
            pltpu.VMEM((120, D), jnp.float32),
            pltpu.VMEM_SHARED((120, D), jnp.float32),
            pltpu.VMEM((C * 4,), jnp.int32),
            pltpu.VMEM((EB,), jnp.int32),
            pltpu.VMEM((EB,), jnp.int32),
            pltpu.VMEM((EB,), jnp.int32),
            pltpu.VMEM((EB,), jnp.int32),
            pltpu.VMEM((EB,), jnp.int32),
            pltpu.VMEM((C, D), jnp.float32),
            pltpu.SemaphoreType.DMA,
        ],
    )
    rows = k(et_flat, bond_flat, stereo_flat, conj_flat, ring_flat)

    transpose = pl.pallas_call(
        _tr_body,
        grid=(N_EDGES // TR_BLK,),
        in_specs=[pl.BlockSpec((TR_BLK, D), lambda i: (i, 0))],
        out_specs=pl.BlockSpec((D, TR_BLK), lambda i: (0, i)),
        out_shape=jax.ShapeDtypeStruct((D, N_EDGES), jnp.float32),
    )
    return transpose(rows)


def kernel(edge_tensor, bond_W, stereo_W, conj_W, ring_W):
    et_flat = (edge_tensor.astype(jnp.int32)
               .reshape(N_EDGES // EB, EB, 4)
               .transpose(2, 0, 1)
               .reshape(-1))
    out_t = _edge_encode(et_flat, bond_W.reshape(-1), stereo_W.reshape(-1),
                         conj_W.reshape(-1), ring_W.reshape(-1))
    return out_t.T


# speedup vs baseline: 5.2997x; 1.8009x over previous
import jax
import jax.numpy as jnp
from jax import lax
from jax.experimental import pallas as pl
from jax.experimental.pallas import tpu as pltpu
from jax.experimental.pallas import tpu_sc as plsc

N_EDGES = 800000
D = 64
L = 16
NC = 2
NS = 16
NW = NC * NS
C = 640
G = 128
NCHUNK = N_EDGES // C
TPW = -(-NCHUNK // NW)


def _sc_body(et_hbm, tabs_hbm, out_hbm,
             tab15_v, bs_v, cr_v, ctab_v, ctab_sh, idx_v, comb0, comb1,
             comb2, comb3, comb4, rows_v, sem):
    comb_k = [comb0, comb1, comb2, comb3, comb4]
    sid = lax.axis_index("s")
    wid = sid * NC + lax.axis_index("c")

    def _build():
        pltpu.sync_copy(tabs_hbm, tab15_v.at[pl.ds(0, 15 * D)])

        def bs_row(r2, _):
            i = r2 // 6
            j = r2 % 6
            for cg in range(D // L):
                o = cg * L
                bs_v[pl.ds(r2 * D + o, L)] = (tab15_v[pl.ds(i * D + o, L)]
                                              + tab15_v[pl.ds((5 + j) * D + o, L)])
            return 0

        lax.fori_loop(0, 30, bs_row, 0)

        for q in range(4):
            for cg in range(D // L):
                o = cg * L
                cr_v[pl.ds(q * D + o, L)] = (tab15_v[pl.ds((11 + q // 2) * D + o, L)]
                                             + tab15_v[pl.ds((13 + q % 2) * D + o, L)])

        def ctab_row(r, _):
            r2 = r // 4
            q = r % 4
            for cg in range(D // L):
                o = cg * L
                ctab_v[r, pl.ds(o, L)] = (bs_v[pl.ds(r2 * D + o, L)]
                                          + cr_v[pl.ds(q * D + o, L)])
            return 0

        lax.fori_loop(0, 120, ctab_row, 0)
        pltpu.sync_copy(ctab_v, ctab_sh)

    _build()
    plsc.subcore_barrier()

    lanes = lax.iota(jnp.int32, L)

    def do_chunk(cid):
        base = cid * C
        for c in range(4):
            pltpu.sync_copy(et_hbm.at[pl.ds(c * N_EDGES + base, C)],
                            idx_v.at[pl.ds(c * C, C)])

        for k in range(C // G):
            def group_t(g2, _, k=k):
                o = k * G + g2 * L
                i0 = idx_v[pl.ds(o, L)]
                i1 = idx_v[pl.ds(C + o, L)]
                i2 = idx_v[pl.ds(2 * C + o, L)]
                i3 = idx_v[pl.ds(3 * C + o, L)]
                comb = (i0 * 24 + i1 * 4 + i2 * 2 + i3) * 4
                comb_k[k][pl.ds(g2 * L, L)] = comb
                return 0

            lax.fori_loop(0, G // L, group_t, 0)

        for k in range(C // G):
            pltpu.async_copy(ctab_sh.at[comb_k[k]],
                             rows_v.at[pl.ds(k * G, G)], sem)
        for k in range(C // G):
            pltpu.make_async_copy(ctab_sh.at[comb_k[k]],
                                  rows_v.at[pl.ds(k * G, G)], sem).wait()

        pltpu.sync_copy(rows_v, out_hbm.at[pl.ds(base, C)])

    def chunk_loop(t, _):
        cid = wid + t * NW
        @pl.when(cid < NCHUNK)
        def _():
            do_chunk(cid)
        return 0

    lax.fori_loop(0, TPW, chunk_loop, 0)


@jax.jit
def _edge_encode(et_flat, tabs_flat):
    mesh = plsc.VectorSubcoreMesh(core_axis_name="c", subcore_axis_name="s")
    k = pl.kernel(
        _sc_body,
        out_type=jax.ShapeDtypeStruct((N_EDGES, D), jnp.float32),
        mesh=mesh,
        compiler_params=pltpu.CompilerParams(needs_layout_passes=False,
                                             disable_bounds_checks=True),
        scratch_types=[
            pltpu.VMEM((16 * D,), jnp.float32),
            pltpu.VMEM((30 * D,), jnp.float32),
            pltpu.VMEM((4 * D,), jnp.float32),
            pltpu.VMEM((120, D), jnp.float32),
            pltpu.VMEM_SHARED((120, D), jnp.float32),
            pltpu.VMEM((C * 4,), jnp.int32),
            pltpu.VMEM((G,), jnp.int32),
            pltpu.VMEM((G,), jnp.int32),
            pltpu.VMEM((G,), jnp.int32),
            pltpu.VMEM((G,), jnp.int32),
            pltpu.VMEM((G,), jnp.int32),
            pltpu.VMEM((C, D), jnp.float32),
            pltpu.SemaphoreType.DMA,
        ],
    )
    return k(et_flat, tabs_flat)


def kernel(edge_tensor, bond_W, stereo_W, conj_W, ring_W):
    et_flat = (edge_tensor.astype(jnp.int32)
               .reshape(N_EDGES // 128, 128, 4)
               .transpose(2, 0, 1)
               .reshape(-1))
    tabs_flat = jnp.concatenate(
        [t[i] for t, n in ((bond_W, 5), (stereo_W, 6), (conj_W, 2),
                           (ring_W, 2)) for i in range(n)], axis=0)
    return _edge_encode(et_flat, tabs_flat)
